# baseline (speedup 1.0000x reference)
import functools

import jax
import jax.numpy as jnp
from jax import lax
from jax.experimental import pallas as pl
from jax.experimental.pallas import tpu as pltpu
from jax.experimental.pallas import tpu_sc as plsc

VOCAB = 1000000
LATENT = 32
BATCH = 16384

NC = 2
NS = 16
NW = NC * NS
L = 16

ROWS = BATCH // NW
VPW = VOCAB // NW
PW = 512
N_PIECES = (VPW + 256 + PW - 1) // PW
MCAP = 4096
WCAP = 256
NCHUNK = WCAP // L
TRASH = BATCH
SROWS = BATCH + 8
BIGV = 1 << 30
TAIL_LO = (VOCAB // 128) * 128
TAIL_W = VOCAB - TAIL_LO
LAST_PIECE = ((VOCAB - TAIL_W - PW) // 128) * 128


def _phase_a(ids_t, table_t, tail_t):
    mesh = plsc.VectorSubcoreMesh(core_axis_name="c", subcore_axis_name="s")

    @functools.partial(
        pl.kernel,
        mesh=mesh,
        compiler_params=pltpu.CompilerParams(
            use_tc_tiling_on_sc=True, needs_layout_passes=False),
        out_type=(
            jax.ShapeDtypeStruct((SROWS, 128), jnp.float32),
            jax.ShapeDtypeStruct((SROWS, 128), jnp.float32),
        ),
        scratch_types=[
            pltpu.VMEM((2, BATCH), jnp.int32),
            pltpu.VMEM((MCAP,), jnp.int32),
            pltpu.VMEM((MCAP,), jnp.int32),
            pltpu.VMEM((MCAP,), jnp.int32),
            pltpu.VMEM((MCAP,), jnp.int32),
            pltpu.VMEM((LATENT, PW), jnp.float32),
            pltpu.VMEM((WCAP,), jnp.int32),
            pltpu.VMEM((WCAP,), jnp.int32),
            pltpu.VMEM((NCHUNK, L), jnp.int32),
            pltpu.VMEM((WCAP, 128), jnp.float32),
            pltpu.VMEM((LATENT, TAIL_W), jnp.float32),
            pltpu.SemaphoreType.DMA,
        ],
    )
    def k(ids_hbm, table_hbm, tail_hbm, s1, s2,
          ids_v, mv1, mp1, mv2, mp2, piece_v, wcol, wpos, wpos2, ebuf,
          tail_v, sem):
        wid = lax.axis_index("s") * NC + lax.axis_index("c")
        lo = wid * VPW
        hi = lo + VPW
        win_lo = (lo // 128) * 128

        pltpu.sync_copy(ids_hbm, ids_v)

        lane = lax.iota(jnp.int32, L)
        big = lane * 0 + BIGV
        for q in range(MCAP // L):
            mv1[pl.ds(q * L, L)] = big
            mv2[pl.ds(q * L, L)] = big

        def compact(m, v, p):
            key = jnp.where(m, lane, lane + L)
            sk, sv = plsc.sort_key_val(key, v)
            _, sp = plsc.sort_key_val(key, p)
            matched = sk < L
            return (jnp.where(matched, sv, BIGV),
                    jnp.where(matched, sp, TRASH),
                    jnp.sum(m.astype(jnp.int32), axis=0))

        def scan_body(kk, offs):
            o1, o2 = offs
            pos = lane + kk * L
            v1 = ids_v[0, pl.ds(kk * L, L)]
            v2 = ids_v[1, pl.ds(kk * L, L)]
            sv1, sp1, c1 = compact((v1 >= lo) & (v1 < hi), v1, pos)
            sv2, sp2, c2 = compact((v2 >= lo) & (v2 < hi), v2, pos)
            mv1[pl.ds(o1, L)] = sv1
            mp1[pl.ds(o1, L)] = sp1
            mv2[pl.ds(o2, L)] = sv2
            mp2[pl.ds(o2, L)] = sp2
            return (o1 + c1, o2 + c2)

        cnt1, cnt2 = lax.fori_loop(0, BATCH // L, scan_body, (0, 0))

        def handle_list(mv, mp, cnt, sdst, piece_lo, pref, pwidth):
            trash = lane * 0 + TRASH
            for q in range(NCHUNK):
                wpos[pl.ds(q * L, L)] = trash

            def sel_body(jj, wn):
                v = mv[pl.ds(jj * L, L)]
                p = mp[pl.ds(jj * L, L)]
                m = (v >= piece_lo) & (v < piece_lo + pwidth)
                key = jnp.where(m, lane, lane + L)
                sk, sc = plsc.sort_key_val(key, v - piece_lo)
                _, sp = plsc.sort_key_val(key, p)
                matched = sk < L
                wcol[pl.ds(wn, L)] = jnp.where(matched, sc, 0)
                wpos[pl.ds(wn, L)] = jnp.where(matched, sp, TRASH)
                return wn + jnp.sum(m.astype(jnp.int32), axis=0)

            nsel = lax.fori_loop(0, (cnt + L - 1) // L, sel_body, 0)

            def chunk_body(cc, _):
                wpos2[cc, pl.ds(0, L)] = wpos[pl.ds(cc * L, L)]
                return 0

            lax.fori_loop(0, (nsel + L - 1) // L, chunk_body, 0)

            def ext_body(kk, _):
                ksplat = lane * 0 + kk
                col = plsc.load_gather(wcol, [ksplat])
                top = plsc.load_gather(pref, [lane, col])
                bot = plsc.load_gather(pref, [lane + L, col])
                ebuf[kk, pl.ds(0, L)] = top
                ebuf[kk, pl.ds(L, L)] = bot
                return 0

            lax.fori_loop(0, nsel, ext_body, 0)

            for q in range(NCHUNK):
                @pl.when(q * L < nsel)
                def _():
                    pltpu.make_async_copy(
                        ebuf.at[pl.ds(q * L, L)],
                        sdst.at[wpos2.at[q]], sem).start()
                    pltpu.make_async_copy(
                        ebuf.at[pl.ds(q * L, L)],
                        sdst.at[wpos2.at[q]], sem).wait()

        def piece_body(pp, _):
            piece_lo = pl.multiple_of(
                jnp.minimum(win_lo + pp * PW, LAST_PIECE), 128)
            cp = pltpu.make_async_copy(
                table_hbm.at[:, pl.ds(piece_lo, PW)], piece_v, sem)
            cp.start()
            cp.wait()
            handle_list(mv1, mp1, cnt1, s1, piece_lo, piece_v, PW)
            handle_list(mv2, mp2, cnt2, s2, piece_lo, piece_v, PW)
            return 0

        lax.fori_loop(0, N_PIECES, piece_body, 0)

        pltpu.sync_copy(tail_hbm, tail_v)
        handle_list(mv1, mp1, cnt1, s1, TAIL_LO, tail_v, TAIL_W)
        handle_list(mv2, mp2, cnt2, s2, TAIL_LO, tail_v, TAIL_W)

    return k(ids_t, table_t, tail_t)


def _phase_b(s1, s2):
    mesh = plsc.VectorSubcoreMesh(core_axis_name="c", subcore_axis_name="s")

    @functools.partial(
        pl.kernel,
        mesh=mesh,
        compiler_params=pltpu.CompilerParams(
            use_tc_tiling_on_sc=True, needs_layout_passes=False),
        out_type=(
            jax.ShapeDtypeStruct((LATENT, BATCH), jnp.float32),
            jax.ShapeDtypeStruct((LATENT, BATCH), jnp.float32),
            jax.ShapeDtypeStruct((LATENT, BATCH), jnp.float32),
        ),
        scratch_types=[
            pltpu.VMEM((ROWS + 8, 128), jnp.float32),
            pltpu.VMEM((LATENT, ROWS), jnp.float32),
            pltpu.VMEM((LATENT, ROWS), jnp.float32),
            pltpu.VMEM((LATENT, ROWS), jnp.float32),
            pltpu.SemaphoreType.DMA,
        ],
    )
    def k(s1_hbm, s2_hbm, out1, out2, out3, b, o1, o2, o3, sem):
        wid = lax.axis_index("s") * NC + lax.axis_index("c")
        base = wid * ROWS
        lookback = lax.rem(base - 8 + BATCH, BATCH)
        lane = lax.iota(jnp.int32, L)

        pltpu.sync_copy(s1_hbm.at[pl.ds(base, ROWS)], b.at[pl.ds(8, ROWS)])

        def l1_body(ll, _):
            lsplat = lane * 0 + ll
            for j in range(ROWS // L):
                r = lane + j * L
                o1[ll, pl.ds(j * L, L)] = plsc.load_gather(b, [r + 8, lsplat])
            return 0

        lax.fori_loop(0, LATENT, l1_body, 0)
        pltpu.sync_copy(o1, out1.at[:, pl.ds(base, ROWS)])

        pltpu.sync_copy(s2_hbm.at[pl.ds(lookback, 8)], b.at[pl.ds(0, 8)])
        pltpu.sync_copy(s2_hbm.at[pl.ds(base, ROWS)], b.at[pl.ds(8, ROWS)])

        def l2_body(ll, _):
            lsplat = lane * 0 + ll
            for j in range(ROWS // L):
                r = lane + j * L
                o2[ll, pl.ds(j * L, L)] = plsc.load_gather(b, [r + 8, lsplat])
                o3[ll, pl.ds(j * L, L)] = plsc.load_gather(b, [r + 7, lsplat])
            return 0

        lax.fori_loop(0, LATENT, l2_body, 0)
        pltpu.sync_copy(o2, out2.at[:, pl.ds(base, ROWS)])
        pltpu.sync_copy(o3, out3.at[:, pl.ds(base, ROWS)])

    return k(s1, s2)


def kernel(input_ids, table):
    ids_t = input_ids.astype(jnp.int32).T
    table_t = table.T
    tail_t = lax.slice(table_t, (0, TAIL_LO), (LATENT, VOCAB))
    s1, s2 = _phase_a(ids_t, table_t, tail_t)
    o1, o2, o3 = _phase_b(s1, s2)
    return (o1.T, o2.T, o3.T)
